# Initial kernel scaffold; baseline (speedup 1.0000x reference)
import jax
import jax.numpy as jnp
from jax.experimental import pallas as pl


def kernel(node_features, edge_index, edge_probs, edge_params):
    raise NotImplementedError("write your pallas kernel here")


# speedup vs baseline: 84.3056x; 84.3056x over previous
import functools

import jax
import jax.numpy as jnp
from jax import lax
from jax.experimental import pallas as pl
from jax.experimental.pallas import tpu as pltpu
from jax.experimental.pallas import tpu_sc as plsc

_NC = 2
_NS = 16
_NW = _NC * _NS
_L = 16


def _sc_partials(n2, ep, p):
    epw = ep // _NW
    nrows = 3 + 2 * p
    mesh = plsc.VectorSubcoreMesh(core_axis_name="c", subcore_axis_name="s")

    @functools.partial(
        pl.kernel,
        out_type=jax.ShapeDtypeStruct((_NW, nrows, _L), jnp.float32),
        mesh=mesh,
        compiler_params=pltpu.CompilerParams(needs_layout_passes=False),
        scratch_types=[
            pltpu.VMEM((n2,), jnp.float32),
            pltpu.VMEM((epw,), jnp.int32),
            pltpu.VMEM((epw,), jnp.int32),
            pltpu.VMEM((epw,), jnp.float32),
            pltpu.VMEM((p * epw,), jnp.float32),
            pltpu.VMEM((nrows, _L), jnp.float32),
        ],
    )
    def sc_kernel(vtab_hbm, src_hbm, dst_hbm, w_hbm, par_hbm, out_hbm,
                  vtab_v, src_v, dst_v, w_v, par_v, res_v):
        wid = lax.axis_index("s") * _NC + lax.axis_index("c")
        base = wid * epw
        pltpu.sync_copy(vtab_hbm, vtab_v)
        pltpu.sync_copy(src_hbm.at[pl.ds(base, epw)], src_v)
        pltpu.sync_copy(dst_hbm.at[pl.ds(base, epw)], dst_v)
        pltpu.sync_copy(w_hbm.at[pl.ds(base, epw)], w_v)
        for j in range(p):
            pltpu.sync_copy(par_hbm.at[pl.ds(j * ep + base, epw)],
                            par_v.at[pl.ds(j * epw, epw)])

        half = jnp.float32(0.5)
        th = jnp.float32(1.5)
        eps = jnp.float32(1e-12)

        def body(g, carry):
            off = g * _L
            s2i = src_v[pl.ds(off, _L)] * 2
            d2i = dst_v[pl.ds(off, _L)] * 2
            vrs = plsc.load_gather(vtab_v, [s2i])
            vis = plsc.load_gather(vtab_v, [s2i + 1])
            vrd = plsc.load_gather(vtab_v, [d2i])
            vid = plsc.load_gather(vtab_v, [d2i + 1])
            w = w_v[pl.ds(off, _L)]
            dr = vrs - vrd
            di = vis - vid
            x = dr * dr + di * di + eps
            yi = 0x5F3759DF - lax.shift_right_logical(
                plsc.bitcast(x, jnp.int32), 1)
            y = plsc.bitcast(yi, jnp.float32)
            hx = half * x
            y = y * (th - hx * y * y)
            y = y * (th - hx * y * y)
            y = y * (th - hx * y * y)
            vd = x * y * w
            wa, t1, t2, s1, s2 = carry
            ns1 = []
            ns2 = []
            for j in range(p):
                pv = par_v[pl.ds(j * epw + off, _L)]
                pw = pv * w
                ns1.append(s1[j] + pw)
                ns2.append(s2[j] + pv * pw)
            return (wa + w, t1 + vd, t2 + vd * vd, tuple(ns1), tuple(ns2))

        zero = jnp.zeros((_L,), jnp.float32)
        init = (zero, zero, zero, (zero,) * p, (zero,) * p)
        wa, t1, t2, s1, s2 = lax.fori_loop(0, epw // _L, body, init)
        res_v[0, :] = wa
        res_v[1, :] = t1
        res_v[2, :] = t2
        for j in range(p):
            res_v[3 + j, :] = s1[j]
            res_v[3 + p + j, :] = s2[j]
        pltpu.sync_copy(res_v, out_hbm.at[wid])

    return sc_kernel


def _tc_finish(e, p, nrows):
    ef = float(e)

    def body(x_ref, o_ref):
        x = x_ref[...]
        s0 = jnp.sum(x[:, 0 * _L:1 * _L])
        t1 = jnp.sum(x[:, 1 * _L:2 * _L])
        t2 = jnp.sum(x[:, 2 * _L:3 * _L])
        denom = s0 + jnp.float32(1e-6)
        acc = jnp.float32(0.0)
        for j in range(p):
            s1 = jnp.sum(x[:, (3 + j) * _L:(4 + j) * _L])
            s2 = jnp.sum(x[:, (3 + p + j) * _L:(4 + p + j) * _L])
            m = s1 / denom
            acc = acc + (s2 - 2.0 * m * s1 + m * m * s0)
        pc = acc / jnp.float32(p)
        vc = (t2 - t1 * t1 / jnp.float32(ef)) / jnp.float32(ef - 1.0)
        o_ref[0, 0] = pc + vc

    return pl.pallas_call(
        body,
        out_shape=jax.ShapeDtypeStruct((1, 1), jnp.float32),
        out_specs=pl.BlockSpec(memory_space=pltpu.SMEM),
    )


def kernel(node_features, edge_index, edge_probs, edge_params):
    n = node_features.shape[0]
    e = edge_index.shape[1]
    p = edge_params.shape[1]
    chunk = _NW * _L
    ep = ((e + chunk - 1) // chunk) * chunk
    pad = ep - e
    src = jnp.pad(edge_index[0], (0, pad))
    dst = jnp.pad(edge_index[1], (0, pad))
    w = jnp.pad(edge_probs, (0, pad))
    par = jnp.pad(edge_params, ((0, pad), (0, 0))).T.reshape(-1)
    vtab = node_features[:, :2].reshape(-1)
    partials = _sc_partials(2 * n, ep, p)(vtab, src, dst, w, par)
    nrows = 3 + 2 * p
    out = _tc_finish(e, p, nrows)(partials.reshape(_NW, nrows * _L))
    return out[0, 0]
